# Initial kernel scaffold; baseline (speedup 1.0000x reference)
import jax
import jax.numpy as jnp
from jax.experimental import pallas as pl


def kernel(x, edge_index, batch, W1_0, W2_0, W1_1, W2_1, W1_2, W2_2, Wc, bc):
    raise NotImplementedError("write your pallas kernel here")


# speedup vs baseline: 6.3232x; 6.3232x over previous
import functools

import jax
import jax.numpy as jnp
from jax import lax
from jax.experimental import pallas as pl
from jax.experimental.pallas import tpu as pltpu
from jax.experimental.pallas import tpu_sc as plsc

N, E, D, H, O, G = 10000, 320000, 128, 128, 64, 64

NC, NS = 2, 16
NW = NC * NS
EPW = E // NW
CK = 80
NCHUNK = EPW // CK
NP_ = 10240
RPT = NP_ // NS


BR = 1000
NB = N // BR


def _sc_agg_build():
    mesh = plsc.VectorSubcoreMesh(
        core_axis_name="c", subcore_axis_name="s",
        num_cores=NC, num_subcores=NS)

    @functools.partial(
        pl.kernel,
        out_type=jax.ShapeDtypeStruct((NC, NP_, H), jnp.float32),
        mesh=mesh,
        scratch_types=[
            pltpu.VMEM((NCHUNK, CK), jnp.int32),
            pltpu.VMEM((NCHUNK, CK), jnp.int32),
            pltpu.VMEM((CK, H), jnp.float32),
            pltpu.VMEM_SHARED((NP_, H), jnp.float32),
            pltpu.SemaphoreType.DMA,
        ],
    )
    def sc_agg(neigh_hbm, row_hbm, col_hbm, zeros_hbm, out_hbm,
               rowv, colv, rows_v, agg_sh, gsem):
        cid = lax.axis_index("c")
        sid = lax.axis_index("s")
        wid = sid * NC + cid

        pltpu.sync_copy(row_hbm.at[wid], rowv)
        pltpu.sync_copy(col_hbm.at[wid], colv)

        pltpu.sync_copy(zeros_hbm, agg_sh.at[pl.ds(sid * RPT, RPT)])
        plsc.subcore_barrier()

        def chunk(i, carry):
            pltpu.async_copy(neigh_hbm.at[colv.at[i]], rows_v, gsem).wait()
            pltpu.sync_copy(rows_v, agg_sh.at[rowv.at[i]], add=True)
            return carry
        lax.fori_loop(0, NCHUNK, chunk, 0)
        plsc.subcore_barrier()

        pltpu.sync_copy(agg_sh.at[pl.ds(sid * RPT, RPT)],
                        out_hbm.at[cid, pl.ds(sid * RPT, RPT)])

    return sc_agg


@functools.cache
def _sc_agg_cached():
    return _sc_agg_build()


def _sc_agg(neigh, row3, col3, zeros2d):
    return _sc_agg_cached()(neigh, row3, col3, zeros2d)


def _mm0_body(x_ref, w_ref, o1_ref, o2_ref):
    out = jnp.dot(x_ref[...], w_ref[...], preferred_element_type=jnp.float32)
    o1_ref[...] = out[:, :H]
    o2_ref[...] = out[:, H:]


def _layer_body(s_ref, a0_ref, a1_ref, w_ref, o1_ref, o2_ref):
    g = jnp.maximum(s_ref[...] + a0_ref[...] + a1_ref[...], 0.0)
    out = jnp.dot(g, w_ref[...], preferred_element_type=jnp.float32)
    o1_ref[...] = out[:, :H]
    o2_ref[...] = out[:, H:]


def _pool_body(s_ref, a0_ref, a1_ref, b_ref, wc_ref, bc_ref, o_ref,
               sums, counts):
    i = pl.program_id(0)

    @pl.when(i == 0)
    def _():
        sums[...] = jnp.zeros_like(sums)
        counts[...] = jnp.zeros_like(counts)

    g = jnp.maximum(s_ref[...] + a0_ref[...] + a1_ref[...], 0.0)
    onehot = (b_ref[...] == lax.broadcasted_iota(jnp.int32, (1, G), 1)
              ).astype(jnp.float32)
    dn = (((0,), (0,)), ((), ()))
    sums[...] += lax.dot_general(onehot, g, dn,
                                 preferred_element_type=jnp.float32)
    counts[...] += lax.dot_general(onehot, jnp.ones((BR, H), jnp.float32), dn,
                                   preferred_element_type=jnp.float32)

    @pl.when(i == NB - 1)
    def _():
        pooled = sums[...] / jnp.maximum(counts[...], 1.0)
        o_ref[...] = jnp.dot(pooled, wc_ref[...],
                             preferred_element_type=jnp.float32) + bc_ref[...]


def _row_spec():
    return pl.BlockSpec((BR, H), lambda i: (i, 0))


def _mm0(x, w12):
    return pl.pallas_call(
        _mm0_body,
        grid=(NB,),
        in_specs=[_row_spec(), pl.BlockSpec((D, 2 * H), lambda i: (0, 0))],
        out_specs=[_row_spec(), _row_spec()],
        out_shape=[jax.ShapeDtypeStruct((N, H), jnp.float32)] * 2,
    )(x, w12)


def _layer(s, a0, a1, w12):
    return pl.pallas_call(
        _layer_body,
        grid=(NB,),
        in_specs=[_row_spec(), _row_spec(), _row_spec(),
                  pl.BlockSpec((H, 2 * H), lambda i: (0, 0))],
        out_specs=[_row_spec(), _row_spec()],
        out_shape=[jax.ShapeDtypeStruct((N, H), jnp.float32)] * 2,
    )(s, a0, a1, w12)


def _pool(s, a0, a1, batch2, wc_t, bc2):
    return pl.pallas_call(
        _pool_body,
        grid=(NB,),
        in_specs=[_row_spec(), _row_spec(), _row_spec(),
                  pl.BlockSpec((BR, 1), lambda i: (i, 0)),
                  pl.BlockSpec((H, O), lambda i: (0, 0)),
                  pl.BlockSpec((1, O), lambda i: (0, 0))],
        out_specs=pl.BlockSpec((G, O), lambda i: (0, 0)),
        out_shape=jax.ShapeDtypeStruct((G, O), jnp.float32),
        scratch_shapes=[pltpu.VMEM((G, H), jnp.float32),
                        pltpu.VMEM((G, H), jnp.float32)],
        compiler_params=pltpu.CompilerParams(
            dimension_semantics=("arbitrary",)),
    )(s, a0, a1, batch2, wc_t, bc2)


def kernel(x, edge_index, batch, W1_0, W2_0, W1_1, W2_1, W1_2, W2_2, Wc, bc):
    w0 = jnp.concatenate([W1_0.T, W2_0.T], axis=1)
    w1 = jnp.concatenate([W1_1.T, W2_1.T], axis=1)
    w2 = jnp.concatenate([W1_2.T, W2_2.T], axis=1)
    row3 = edge_index[0].reshape(NW, NCHUNK, CK)
    col3 = edge_index[1].reshape(NW, NCHUNK, CK)
    batch2 = batch.reshape(N, 1)
    bc2 = bc.reshape(1, O)
    zeros2d = jnp.zeros((RPT, H), jnp.float32)

    s, nb = _mm0(x, w0)
    parts = _sc_agg(nb, row3, col3, zeros2d)
    s, nb = _layer(s, parts[0], parts[1], w1)
    parts = _sc_agg(nb, row3, col3, zeros2d)
    s, nb = _layer(s, parts[0], parts[1], w2)
    parts = _sc_agg(nb, row3, col3, zeros2d)
    return _pool(s, parts[0], parts[1], batch2, Wc.T, bc2)


# speedup vs baseline: 10.4027x; 1.6452x over previous
import functools

import jax
import jax.numpy as jnp
from jax import lax
from jax.experimental import pallas as pl
from jax.experimental.pallas import tpu as pltpu
from jax.experimental.pallas import tpu_sc as plsc

N, E, D, H, O, G = 10000, 320000, 128, 128, 64, 64

NC, NS = 2, 16
HH = H // NC
EPT = E // NS
CK = 80
NCHUNK = EPT // CK
NBUF = 5
NP_ = 10240
RPT = NP_ // NS

BR = 1000
NB = N // BR


def _sc_agg_build():
    mesh = plsc.VectorSubcoreMesh(
        core_axis_name="c", subcore_axis_name="s",
        num_cores=NC, num_subcores=NS)

    @functools.partial(
        pl.kernel,
        out_type=jax.ShapeDtypeStruct((NC, NP_, HH), jnp.float32),
        mesh=mesh,
        scratch_types=[
            pltpu.VMEM((NCHUNK, CK), jnp.int32),
            pltpu.VMEM((NCHUNK, CK), jnp.int32),
            [pltpu.VMEM((CK, HH), jnp.float32)] * NBUF,
            [pltpu.SemaphoreType.DMA] * NBUF,
            pltpu.VMEM_SHARED((NP_, HH), jnp.float32),
        ],
        compiler_params=pltpu.CompilerParams(use_tc_tiling_on_sc=False),
    )
    def sc_agg(neigh_hbm, row_hbm, col_hbm, zeros_hbm, out_hbm,
               rowv, colv, bufs, gsems, agg_sh):
        cid = lax.axis_index("c")
        sid = lax.axis_index("s")

        pltpu.sync_copy(row_hbm.at[sid], rowv)
        pltpu.sync_copy(col_hbm.at[cid, sid], colv)

        for b in range(NBUF):
            pltpu.async_copy(neigh_hbm.at[colv.at[b]], bufs[b], gsems[b])

        pltpu.sync_copy(zeros_hbm, agg_sh.at[pl.ds(sid * RPT, RPT)])
        plsc.subcore_barrier()

        def block(t, carry):
            for b in range(NBUF):
                i = t * NBUF + b
                pltpu.make_async_copy(
                    neigh_hbm.at[colv.at[i]], bufs[b], gsems[b]).wait()
                pltpu.sync_copy(bufs[b], agg_sh.at[rowv.at[i]], add=True)
                pltpu.async_copy(
                    neigh_hbm.at[colv.at[i + NBUF]], bufs[b], gsems[b])
            return carry
        lax.fori_loop(0, NCHUNK // NBUF - 1, block, 0)
        for b in range(NBUF):
            i = NCHUNK - NBUF + b
            pltpu.make_async_copy(
                neigh_hbm.at[colv.at[i]], bufs[b], gsems[b]).wait()
            pltpu.sync_copy(bufs[b], agg_sh.at[rowv.at[i]], add=True)
        plsc.subcore_barrier()

        pltpu.sync_copy(agg_sh.at[pl.ds(sid * RPT, RPT)],
                        out_hbm.at[cid, pl.ds(sid * RPT, RPT)])

    return sc_agg


@functools.cache
def _sc_agg_cached():
    return _sc_agg_build()


def _sc_agg(neigh_flat, row3, col4, zeros2d):
    return _sc_agg_cached()(neigh_flat, row3, col4, zeros2d)


def _mm0_body(x_ref, w_ref, o1_ref, o2_ref):
    out = jnp.dot(x_ref[...], w_ref[...], preferred_element_type=jnp.float32)
    o1_ref[...] = out[:, :H]
    o2_ref[...] = jnp.stack([out[:, H:H + HH], out[:, H + HH:]], axis=0)


def _layer_body(s_ref, a0_ref, a1_ref, w_ref, o1_ref, o2_ref):
    agg = jnp.concatenate([a0_ref[...], a1_ref[...]], axis=1)
    g = jnp.maximum(s_ref[...] + agg, 0.0)
    out = jnp.dot(g, w_ref[...], preferred_element_type=jnp.float32)
    o1_ref[...] = out[:, :H]
    o2_ref[...] = jnp.stack([out[:, H:H + HH], out[:, H + HH:]], axis=0)


def _pool_body(s_ref, a0_ref, a1_ref, b_ref, wc_ref, bc_ref, o_ref,
               sums, counts):
    i = pl.program_id(0)

    @pl.when(i == 0)
    def _():
        sums[...] = jnp.zeros_like(sums)
        counts[...] = jnp.zeros_like(counts)

    agg = jnp.concatenate([a0_ref[...], a1_ref[...]], axis=1)
    g = jnp.maximum(s_ref[...] + agg, 0.0)
    onehot = (b_ref[...] == lax.broadcasted_iota(jnp.int32, (1, G), 1)
              ).astype(jnp.float32)
    dn = (((0,), (0,)), ((), ()))
    sums[...] += lax.dot_general(onehot, g, dn,
                                 preferred_element_type=jnp.float32)
    counts[...] += lax.dot_general(onehot, jnp.ones((BR, H), jnp.float32), dn,
                                   preferred_element_type=jnp.float32)

    @pl.when(i == NB - 1)
    def _():
        pooled = sums[...] / jnp.maximum(counts[...], 1.0)
        o_ref[...] = jnp.dot(pooled, wc_ref[...],
                             preferred_element_type=jnp.float32) + bc_ref[...]


def _row_spec():
    return pl.BlockSpec((BR, H), lambda i: (i, 0))


def _half_spec():
    return pl.BlockSpec((BR, HH), lambda i: (i, 0))


def _nb2_spec():
    return pl.BlockSpec((2, BR, HH), lambda i: (0, i, 0))


def _mm0(x, w12):
    return pl.pallas_call(
        _mm0_body,
        grid=(NB,),
        in_specs=[_row_spec(), pl.BlockSpec((D, 2 * H), lambda i: (0, 0))],
        out_specs=[_row_spec(), _nb2_spec()],
        out_shape=[jax.ShapeDtypeStruct((N, H), jnp.float32),
                   jax.ShapeDtypeStruct((2, N, HH), jnp.float32)],
    )(x, w12)


def _layer(s, a0, a1, w12):
    return pl.pallas_call(
        _layer_body,
        grid=(NB,),
        in_specs=[_row_spec(), _half_spec(), _half_spec(),
                  pl.BlockSpec((H, 2 * H), lambda i: (0, 0))],
        out_specs=[_row_spec(), _nb2_spec()],
        out_shape=[jax.ShapeDtypeStruct((N, H), jnp.float32),
                   jax.ShapeDtypeStruct((2, N, HH), jnp.float32)],
    )(s, a0, a1, w12)


def _pool(s, a0, a1, batch2, wc_t, bc2):
    return pl.pallas_call(
        _pool_body,
        grid=(NB,),
        in_specs=[_row_spec(), _half_spec(), _half_spec(),
                  pl.BlockSpec((BR, 1), lambda i: (i, 0)),
                  pl.BlockSpec((H, O), lambda i: (0, 0)),
                  pl.BlockSpec((1, O), lambda i: (0, 0))],
        out_specs=pl.BlockSpec((G, O), lambda i: (0, 0)),
        out_shape=jax.ShapeDtypeStruct((G, O), jnp.float32),
        scratch_shapes=[pltpu.VMEM((G, H), jnp.float32),
                        pltpu.VMEM((G, H), jnp.float32)],
        compiler_params=pltpu.CompilerParams(
            dimension_semantics=("arbitrary",)),
    )(s, a0, a1, batch2, wc_t, bc2)


def kernel(x, edge_index, batch, W1_0, W2_0, W1_1, W2_1, W1_2, W2_2, Wc, bc):
    w0 = jnp.concatenate([W1_0.T, W2_0.T], axis=1)
    w1 = jnp.concatenate([W1_1.T, W2_1.T], axis=1)
    w2 = jnp.concatenate([W1_2.T, W2_2.T], axis=1)
    row3 = edge_index[0].reshape(NS, NCHUNK, CK)
    col = edge_index[1]
    col4 = jnp.stack([col, col + N]).reshape(NC, NS, NCHUNK, CK)
    batch2 = batch.reshape(N, 1)
    bc2 = bc.reshape(1, O)
    zeros2d = jnp.zeros((RPT, HH), jnp.float32)

    s, nb2 = _mm0(x, w0)
    parts = _sc_agg(nb2.reshape(2 * N, HH), row3, col4, zeros2d)
    s, nb2 = _layer(s, parts[0], parts[1], w1)
    parts = _sc_agg(nb2.reshape(2 * N, HH), row3, col4, zeros2d)
    s, nb2 = _layer(s, parts[0], parts[1], w2)
    parts = _sc_agg(nb2.reshape(2 * N, HH), row3, col4, zeros2d)
    return _pool(s, parts[0], parts[1], batch2, Wc.T, bc2)


# speedup vs baseline: 10.6543x; 1.0242x over previous
import functools

import jax
import jax.numpy as jnp
from jax import lax
from jax.experimental import pallas as pl
from jax.experimental.pallas import tpu as pltpu
from jax.experimental.pallas import tpu_sc as plsc

N, E, D, H, O, G = 10000, 320000, 128, 128, 64, 64

NC, NS = 2, 16
HH = H // NC
EPT = E // NS
CK = 80
NCHUNK = EPT // CK
NBUF = 5
NP_ = 10240
RPT = NP_ // NS

BR = 1000
NB = N // BR


def _sc_agg_build():
    mesh = plsc.VectorSubcoreMesh(
        core_axis_name="c", subcore_axis_name="s",
        num_cores=NC, num_subcores=NS)

    @functools.partial(
        pl.kernel,
        out_type=jax.ShapeDtypeStruct((NC, NP_, HH), jnp.float32),
        mesh=mesh,
        scratch_types=[
            pltpu.VMEM((NCHUNK, CK), jnp.int32),
            pltpu.VMEM((NCHUNK, CK), jnp.int32),
            [pltpu.VMEM((CK, HH), jnp.float32)] * NBUF,
            [pltpu.SemaphoreType.DMA] * NBUF,
            pltpu.VMEM_SHARED((NP_, HH), jnp.float32),
        ],
        compiler_params=pltpu.CompilerParams(use_tc_tiling_on_sc=False),
    )
    def sc_agg(neigh_hbm, row_hbm, col_hbm, zeros_hbm, out_hbm,
               rowv, colv, bufs, gsems, agg_sh):
        cid = lax.axis_index("c")
        sid = lax.axis_index("s")

        pltpu.sync_copy(row_hbm.at[sid], rowv)
        pltpu.sync_copy(col_hbm.at[sid], colv)
        nh = neigh_hbm.at[cid]

        for b in range(NBUF):
            pltpu.async_copy(nh.at[colv.at[b]], bufs[b], gsems[b])

        pltpu.sync_copy(zeros_hbm, agg_sh.at[pl.ds(sid * RPT, RPT)])
        plsc.subcore_barrier()

        def block(t, carry):
            for b in range(NBUF):
                i = t * NBUF + b
                pltpu.make_async_copy(
                    nh.at[colv.at[i]], bufs[b], gsems[b]).wait()
                pltpu.sync_copy(bufs[b], agg_sh.at[rowv.at[i]], add=True)
                pltpu.async_copy(
                    nh.at[colv.at[i + NBUF]], bufs[b], gsems[b])
            return carry
        lax.fori_loop(0, NCHUNK // NBUF - 1, block, 0)
        for b in range(NBUF):
            i = NCHUNK - NBUF + b
            pltpu.make_async_copy(
                nh.at[colv.at[i]], bufs[b], gsems[b]).wait()
            pltpu.sync_copy(bufs[b], agg_sh.at[rowv.at[i]], add=True)
        plsc.subcore_barrier()

        pltpu.sync_copy(agg_sh.at[pl.ds(sid * RPT, RPT)],
                        out_hbm.at[cid, pl.ds(sid * RPT, RPT)])

    return sc_agg


@functools.cache
def _sc_agg_cached():
    return _sc_agg_build()


def _sc_agg(neigh2, row3, col3, zeros2d):
    return _sc_agg_cached()(neigh2, row3, col3, zeros2d)


def _mm0_body(x_ref, w_ref, o1_ref, o2_ref):
    out = jnp.dot(x_ref[...], w_ref[...], preferred_element_type=jnp.float32)
    o1_ref[...] = out[:, :H]
    o2_ref[...] = jnp.stack([out[:, H:H + HH], out[:, H + HH:]], axis=0)


def _layer_body(s_ref, a0_ref, a1_ref, w_ref, o1_ref, o2_ref):
    agg = jnp.concatenate([a0_ref[...], a1_ref[...]], axis=1)
    g = jnp.maximum(s_ref[...] + agg, 0.0)
    out = jnp.dot(g, w_ref[...], preferred_element_type=jnp.float32)
    o1_ref[...] = out[:, :H]
    o2_ref[...] = jnp.stack([out[:, H:H + HH], out[:, H + HH:]], axis=0)


def _pool_body(s_ref, a0_ref, a1_ref, b_ref, wc_ref, bc_ref, o_ref,
               sums, counts):
    i = pl.program_id(0)

    @pl.when(i == 0)
    def _():
        sums[...] = jnp.zeros_like(sums)
        counts[...] = jnp.zeros_like(counts)

    agg = jnp.concatenate([a0_ref[...], a1_ref[...]], axis=1)
    g = jnp.maximum(s_ref[...] + agg, 0.0)
    onehot = (b_ref[...] == lax.broadcasted_iota(jnp.int32, (1, G), 1)
              ).astype(jnp.float32)
    dn = (((0,), (0,)), ((), ()))
    sums[...] += lax.dot_general(onehot, g, dn,
                                 preferred_element_type=jnp.float32)
    counts[...] += lax.dot_general(onehot, jnp.ones((BR, H), jnp.float32), dn,
                                   preferred_element_type=jnp.float32)

    @pl.when(i == NB - 1)
    def _():
        pooled = sums[...] / jnp.maximum(counts[...], 1.0)
        o_ref[...] = jnp.dot(pooled, wc_ref[...],
                             preferred_element_type=jnp.float32) + bc_ref[...]


def _row_spec():
    return pl.BlockSpec((BR, H), lambda i: (i, 0))


def _half_spec():
    return pl.BlockSpec((BR, HH), lambda i: (i, 0))


def _nb2_spec():
    return pl.BlockSpec((2, BR, HH), lambda i: (0, i, 0))


def _mm0(x, w12):
    return pl.pallas_call(
        _mm0_body,
        grid=(NB,),
        in_specs=[_row_spec(), pl.BlockSpec((D, 2 * H), lambda i: (0, 0))],
        out_specs=[_row_spec(), _nb2_spec()],
        out_shape=[jax.ShapeDtypeStruct((N, H), jnp.float32),
                   jax.ShapeDtypeStruct((2, N, HH), jnp.float32)],
    )(x, w12)


def _layer(s, a0, a1, w12):
    return pl.pallas_call(
        _layer_body,
        grid=(NB,),
        in_specs=[_row_spec(), _half_spec(), _half_spec(),
                  pl.BlockSpec((H, 2 * H), lambda i: (0, 0))],
        out_specs=[_row_spec(), _nb2_spec()],
        out_shape=[jax.ShapeDtypeStruct((N, H), jnp.float32),
                   jax.ShapeDtypeStruct((2, N, HH), jnp.float32)],
    )(s, a0, a1, w12)


def _pool(s, a0, a1, batch2, wc_t, bc2):
    return pl.pallas_call(
        _pool_body,
        grid=(NB,),
        in_specs=[_row_spec(), _half_spec(), _half_spec(),
                  pl.BlockSpec((BR, 1), lambda i: (i, 0)),
                  pl.BlockSpec((H, O), lambda i: (0, 0)),
                  pl.BlockSpec((1, O), lambda i: (0, 0))],
        out_specs=pl.BlockSpec((G, O), lambda i: (0, 0)),
        out_shape=jax.ShapeDtypeStruct((G, O), jnp.float32),
        scratch_shapes=[pltpu.VMEM((G, H), jnp.float32),
                        pltpu.VMEM((G, H), jnp.float32)],
        compiler_params=pltpu.CompilerParams(
            dimension_semantics=("arbitrary",)),
    )(s, a0, a1, batch2, wc_t, bc2)


def kernel(x, edge_index, batch, W1_0, W2_0, W1_1, W2_1, W1_2, W2_2, Wc, bc):
    w0 = jnp.concatenate([W1_0.T, W2_0.T], axis=1)
    w1 = jnp.concatenate([W1_1.T, W2_1.T], axis=1)
    w2 = jnp.concatenate([W1_2.T, W2_2.T], axis=1)
    row3 = edge_index[0].reshape(NS, NCHUNK, CK)
    col3 = edge_index[1].reshape(NS, NCHUNK, CK)
    batch2 = batch.reshape(N, 1)
    bc2 = bc.reshape(1, O)
    zeros2d = jnp.zeros((RPT, HH), jnp.float32)

    s, nb2 = _mm0(x, w0)
    parts = _sc_agg(nb2, row3, col3, zeros2d)
    s, nb2 = _layer(s, parts[0], parts[1], w1)
    parts = _sc_agg(nb2, row3, col3, zeros2d)
    s, nb2 = _layer(s, parts[0], parts[1], w2)
    parts = _sc_agg(nb2, row3, col3, zeros2d)
    return _pool(s, parts[0], parts[1], batch2, Wc.T, bc2)


# speedup vs baseline: 11.2898x; 1.0596x over previous
import functools

import jax
import jax.numpy as jnp
from jax import lax
from jax.experimental import pallas as pl
from jax.experimental.pallas import tpu as pltpu
from jax.experimental.pallas import tpu_sc as plsc

N, E, D, H, O, G = 10000, 320000, 128, 128, 64, 64

NC, NS = 2, 16
HH = H // NC
EPT = E // NS
CK = 80
NCHUNK = EPT // CK
NBUF = 5
NP_ = 10240
RPT = NP_ // NS

BR = 1000
NB = N // BR


def _sc_agg_build():
    mesh = plsc.VectorSubcoreMesh(
        core_axis_name="c", subcore_axis_name="s",
        num_cores=NC, num_subcores=NS)

    @functools.partial(
        pl.kernel,
        out_type=jax.ShapeDtypeStruct((NC, NP_, HH), jnp.float32),
        mesh=mesh,
        scratch_types=[
            pltpu.VMEM((NCHUNK, CK), jnp.int32),
            pltpu.VMEM((NCHUNK, CK), jnp.int32),
            [pltpu.VMEM((CK, HH), jnp.float32)] * NBUF,
            [pltpu.SemaphoreType.DMA] * NBUF,
            pltpu.VMEM_SHARED((NP_, HH), jnp.float32),
        ],
        compiler_params=pltpu.CompilerParams(use_tc_tiling_on_sc=False),
    )
    def sc_agg(neigh_hbm, row_hbm, col_hbm, zeros_hbm, out_hbm,
               rowv, colv, bufs, gsems, agg_sh):
        cid = lax.axis_index("c")
        sid = lax.axis_index("s")

        pltpu.sync_copy(row_hbm.at[sid], rowv)
        pltpu.sync_copy(col_hbm.at[sid], colv)
        nh = neigh_hbm.at[cid]

        for b in range(NBUF):
            pltpu.async_copy(nh.at[colv.at[b]], bufs[b], gsems[b])

        pltpu.sync_copy(zeros_hbm, agg_sh.at[pl.ds(sid * RPT, RPT)])
        plsc.subcore_barrier()

        def block(t, carry):
            for b in range(NBUF):
                i = t * NBUF + b
                pltpu.make_async_copy(
                    nh.at[colv.at[i]], bufs[b], gsems[b]).wait()
                pltpu.sync_copy(bufs[b], agg_sh.at[rowv.at[i]], add=True)
                pltpu.async_copy(
                    nh.at[colv.at[i + NBUF]], bufs[b], gsems[b])
            return carry
        lax.fori_loop(0, NCHUNK // NBUF - 1, block, 0)
        for b in range(NBUF):
            i = NCHUNK - NBUF + b
            pltpu.make_async_copy(
                nh.at[colv.at[i]], bufs[b], gsems[b]).wait()
            pltpu.sync_copy(bufs[b], agg_sh.at[rowv.at[i]], add=True)
        plsc.subcore_barrier()

        pltpu.sync_copy(agg_sh.at[pl.ds(sid * RPT, RPT)],
                        out_hbm.at[cid, pl.ds(sid * RPT, RPT)])

    return sc_agg


@functools.cache
def _sc_agg_cached():
    return _sc_agg_build()


def _sc_agg(neigh2, row3, col3, zeros2d):
    return _sc_agg_cached()(neigh2, row3, col3, zeros2d)


def _mm0_body(x_ref, w_ref, o1_ref, o2_ref):
    out = jnp.dot(x_ref[...], w_ref[...], preferred_element_type=jnp.float32)
    o1_ref[...] = out[:, :H]
    o2_ref[...] = jnp.stack([out[:, H:H + HH], out[:, H + HH:]], axis=0)


def _layer_body(s_ref, p_ref, w_ref, o1_ref, o2_ref):
    agg = jnp.concatenate([p_ref[0], p_ref[1]], axis=1)
    g = jnp.maximum(s_ref[...] + agg, 0.0)
    out = jnp.dot(g, w_ref[...], preferred_element_type=jnp.float32)
    o1_ref[...] = out[:, :H]
    o2_ref[...] = jnp.stack([out[:, H:H + HH], out[:, H + HH:]], axis=0)


def _pool_body(s_ref, p_ref, b_ref, wc_ref, bc_ref, o_ref,
               sums, counts):
    i = pl.program_id(0)

    @pl.when(i == 0)
    def _():
        sums[...] = jnp.zeros_like(sums)
        counts[...] = jnp.zeros_like(counts)

    agg = jnp.concatenate([p_ref[0], p_ref[1]], axis=1)
    g = jnp.maximum(s_ref[...] + agg, 0.0)
    onehot = (b_ref[...] == lax.broadcasted_iota(jnp.int32, (1, G), 1)
              ).astype(jnp.float32)
    dn = (((0,), (0,)), ((), ()))
    sums[...] += lax.dot_general(onehot, g, dn,
                                 preferred_element_type=jnp.float32)
    counts[...] += lax.dot_general(onehot, jnp.ones((BR, H), jnp.float32), dn,
                                   preferred_element_type=jnp.float32)

    @pl.when(i == NB - 1)
    def _():
        pooled = sums[...] / jnp.maximum(counts[...], 1.0)
        o_ref[...] = jnp.dot(pooled, wc_ref[...],
                             preferred_element_type=jnp.float32) + bc_ref[...]


def _row_spec():
    return pl.BlockSpec((BR, H), lambda i: (i, 0))


def _parts_spec():
    return pl.BlockSpec((NC, BR, HH), lambda i: (0, i, 0))


def _nb2_spec():
    return pl.BlockSpec((2, BR, HH), lambda i: (0, i, 0))


def _mm0(x, w12):
    return pl.pallas_call(
        _mm0_body,
        grid=(NB,),
        in_specs=[_row_spec(), pl.BlockSpec((D, 2 * H), lambda i: (0, 0))],
        out_specs=[_row_spec(), _nb2_spec()],
        out_shape=[jax.ShapeDtypeStruct((N, H), jnp.float32),
                   jax.ShapeDtypeStruct((2, N, HH), jnp.float32)],
    )(x, w12)


def _layer(s, parts, w12):
    return pl.pallas_call(
        _layer_body,
        grid=(NB,),
        in_specs=[_row_spec(), _parts_spec(),
                  pl.BlockSpec((H, 2 * H), lambda i: (0, 0))],
        out_specs=[_row_spec(), _nb2_spec()],
        out_shape=[jax.ShapeDtypeStruct((N, H), jnp.float32),
                   jax.ShapeDtypeStruct((2, N, HH), jnp.float32)],
    )(s, parts, w12)


def _pool(s, parts, batch2, wc_t, bc2):
    return pl.pallas_call(
        _pool_body,
        grid=(NB,),
        in_specs=[_row_spec(), _parts_spec(),
                  pl.BlockSpec((BR, 1), lambda i: (i, 0)),
                  pl.BlockSpec((H, O), lambda i: (0, 0)),
                  pl.BlockSpec((1, O), lambda i: (0, 0))],
        out_specs=pl.BlockSpec((G, O), lambda i: (0, 0)),
        out_shape=jax.ShapeDtypeStruct((G, O), jnp.float32),
        scratch_shapes=[pltpu.VMEM((G, H), jnp.float32),
                        pltpu.VMEM((G, H), jnp.float32)],
        compiler_params=pltpu.CompilerParams(
            dimension_semantics=("arbitrary",)),
    )(s, parts, batch2, wc_t, bc2)


def kernel(x, edge_index, batch, W1_0, W2_0, W1_1, W2_1, W1_2, W2_2, Wc, bc):
    w0 = jnp.concatenate([W1_0.T, W2_0.T], axis=1)
    w1 = jnp.concatenate([W1_1.T, W2_1.T], axis=1)
    w2 = jnp.concatenate([W1_2.T, W2_2.T], axis=1)
    row3 = edge_index[0].reshape(NS, NCHUNK, CK)
    col3 = edge_index[1].reshape(NS, NCHUNK, CK)
    batch2 = batch.reshape(N, 1)
    bc2 = bc.reshape(1, O)
    zeros2d = jnp.zeros((RPT, HH), jnp.float32)

    s, nb2 = _mm0(x, w0)
    parts = _sc_agg(nb2, row3, col3, zeros2d)
    s, nb2 = _layer(s, parts, w1)
    parts = _sc_agg(nb2, row3, col3, zeros2d)
    s, nb2 = _layer(s, parts, w2)
    parts = _sc_agg(nb2, row3, col3, zeros2d)
    return _pool(s, parts, batch2, Wc.T, bc2)


# speedup vs baseline: 11.5614x; 1.0241x over previous
import functools

import jax
import jax.numpy as jnp
from jax import lax
from jax.experimental import pallas as pl
from jax.experimental.pallas import tpu as pltpu
from jax.experimental.pallas import tpu_sc as plsc

N, E, D, H, O, G = 10000, 320000, 128, 128, 64, 64

NC, NS = 2, 16
HH = H // NC
EPT = E // NS
CK = 80
NCHUNK = EPT // CK
NBUF = 5
NP_ = 10240
RPT = NP_ // NS

BR = 2000
NB = N // BR


def _sc_agg_build():
    mesh = plsc.VectorSubcoreMesh(
        core_axis_name="c", subcore_axis_name="s",
        num_cores=NC, num_subcores=NS)

    @functools.partial(
        pl.kernel,
        out_type=jax.ShapeDtypeStruct((NC, NP_, HH), jnp.float32),
        mesh=mesh,
        scratch_types=[
            pltpu.VMEM((NCHUNK, CK), jnp.int32),
            pltpu.VMEM((NCHUNK, CK), jnp.int32),
            [pltpu.VMEM((CK, HH), jnp.float32)] * NBUF,
            [pltpu.SemaphoreType.DMA] * NBUF,
            pltpu.VMEM_SHARED((NP_, HH), jnp.float32),
        ],
        compiler_params=pltpu.CompilerParams(use_tc_tiling_on_sc=False),
    )
    def sc_agg(neigh_hbm, row_hbm, col_hbm, zeros_hbm, out_hbm,
               rowv, colv, bufs, gsems, agg_sh):
        cid = lax.axis_index("c")
        sid = lax.axis_index("s")

        pltpu.sync_copy(row_hbm.at[sid], rowv)
        pltpu.sync_copy(col_hbm.at[sid], colv)
        nh = neigh_hbm.at[cid]

        for b in range(NBUF):
            pltpu.async_copy(nh.at[colv.at[b]], bufs[b], gsems[b])

        pltpu.sync_copy(zeros_hbm, agg_sh.at[pl.ds(sid * RPT, RPT)])
        plsc.subcore_barrier()

        def block(t, carry):
            for b in range(NBUF):
                i = t * NBUF + b
                pltpu.make_async_copy(
                    nh.at[colv.at[i]], bufs[b], gsems[b]).wait()
                pltpu.sync_copy(bufs[b], agg_sh.at[rowv.at[i]], add=True)
                pltpu.async_copy(
                    nh.at[colv.at[i + NBUF]], bufs[b], gsems[b])
            return carry
        lax.fori_loop(0, NCHUNK // NBUF - 1, block, 0)
        for b in range(NBUF):
            i = NCHUNK - NBUF + b
            pltpu.make_async_copy(
                nh.at[colv.at[i]], bufs[b], gsems[b]).wait()
            pltpu.sync_copy(bufs[b], agg_sh.at[rowv.at[i]], add=True)
        plsc.subcore_barrier()

        pltpu.sync_copy(agg_sh.at[pl.ds(sid * RPT, RPT)],
                        out_hbm.at[cid, pl.ds(sid * RPT, RPT)])

    return sc_agg


@functools.cache
def _sc_agg_cached():
    return _sc_agg_build()


def _sc_agg(neigh2, row3, col3, zeros2d):
    return _sc_agg_cached()(neigh2, row3, col3, zeros2d)


def _mm0_body(x_ref, w_ref, o1_ref, o2_ref):
    out = jnp.dot(x_ref[...], w_ref[...], preferred_element_type=jnp.float32)
    o1_ref[...] = out[:, :H]
    o2_ref[...] = jnp.stack([out[:, H:H + HH], out[:, H + HH:]], axis=0)


def _layer_body(s_ref, p_ref, w_ref, o1_ref, o2_ref):
    agg = jnp.concatenate([p_ref[0], p_ref[1]], axis=1)
    g = jnp.maximum(s_ref[...] + agg, 0.0)
    out = jnp.dot(g, w_ref[...], preferred_element_type=jnp.float32)
    o1_ref[...] = out[:, :H]
    o2_ref[...] = jnp.stack([out[:, H:H + HH], out[:, H + HH:]], axis=0)


def _pool_body(s_ref, p_ref, b_ref, wc_ref, bc_ref, o_ref,
               sums, counts):
    i = pl.program_id(0)

    @pl.when(i == 0)
    def _():
        sums[...] = jnp.zeros_like(sums)
        counts[...] = jnp.zeros_like(counts)

    agg = jnp.concatenate([p_ref[0], p_ref[1]], axis=1)
    g = jnp.maximum(s_ref[...] + agg, 0.0)
    onehot = (b_ref[...] == lax.broadcasted_iota(jnp.int32, (1, G), 1)
              ).astype(jnp.float32)
    dn = (((0,), (0,)), ((), ()))
    sums[...] += lax.dot_general(onehot, g, dn,
                                 preferred_element_type=jnp.float32)
    counts[...] += lax.dot_general(onehot, jnp.ones((BR, H), jnp.float32), dn,
                                   preferred_element_type=jnp.float32)

    @pl.when(i == NB - 1)
    def _():
        pooled = sums[...] / jnp.maximum(counts[...], 1.0)
        o_ref[...] = jnp.dot(pooled, wc_ref[...],
                             preferred_element_type=jnp.float32) + bc_ref[...]


def _row_spec():
    return pl.BlockSpec((BR, H), lambda i: (i, 0))


def _parts_spec():
    return pl.BlockSpec((NC, BR, HH), lambda i: (0, i, 0))


def _nb2_spec():
    return pl.BlockSpec((2, BR, HH), lambda i: (0, i, 0))


def _mm0(x, w12):
    return pl.pallas_call(
        _mm0_body,
        grid=(NB,),
        in_specs=[_row_spec(), pl.BlockSpec((D, 2 * H), lambda i: (0, 0))],
        out_specs=[_row_spec(), _nb2_spec()],
        out_shape=[jax.ShapeDtypeStruct((N, H), jnp.float32),
                   jax.ShapeDtypeStruct((2, N, HH), jnp.float32)],
    )(x, w12)


def _layer(s, parts, w12):
    return pl.pallas_call(
        _layer_body,
        grid=(NB,),
        in_specs=[_row_spec(), _parts_spec(),
                  pl.BlockSpec((H, 2 * H), lambda i: (0, 0))],
        out_specs=[_row_spec(), _nb2_spec()],
        out_shape=[jax.ShapeDtypeStruct((N, H), jnp.float32),
                   jax.ShapeDtypeStruct((2, N, HH), jnp.float32)],
    )(s, parts, w12)


def _pool(s, parts, batch2, wc_t, bc2):
    return pl.pallas_call(
        _pool_body,
        grid=(NB,),
        in_specs=[_row_spec(), _parts_spec(),
                  pl.BlockSpec((BR, 1), lambda i: (i, 0)),
                  pl.BlockSpec((H, O), lambda i: (0, 0)),
                  pl.BlockSpec((1, O), lambda i: (0, 0))],
        out_specs=pl.BlockSpec((G, O), lambda i: (0, 0)),
        out_shape=jax.ShapeDtypeStruct((G, O), jnp.float32),
        scratch_shapes=[pltpu.VMEM((G, H), jnp.float32),
                        pltpu.VMEM((G, H), jnp.float32)],
        compiler_params=pltpu.CompilerParams(
            dimension_semantics=("arbitrary",)),
    )(s, parts, batch2, wc_t, bc2)


def kernel(x, edge_index, batch, W1_0, W2_0, W1_1, W2_1, W1_2, W2_2, Wc, bc):
    w0 = jnp.concatenate([W1_0.T, W2_0.T], axis=1)
    w1 = jnp.concatenate([W1_1.T, W2_1.T], axis=1)
    w2 = jnp.concatenate([W1_2.T, W2_2.T], axis=1)
    row3 = edge_index[0].reshape(NS, NCHUNK, CK)
    col3 = edge_index[1].reshape(NS, NCHUNK, CK)
    batch2 = batch.reshape(N, 1)
    bc2 = bc.reshape(1, O)
    zeros2d = jnp.zeros((RPT, HH), jnp.float32)

    s, nb2 = _mm0(x, w0)
    parts = _sc_agg(nb2, row3, col3, zeros2d)
    s, nb2 = _layer(s, parts, w1)
    parts = _sc_agg(nb2, row3, col3, zeros2d)
    s, nb2 = _layer(s, parts, w2)
    parts = _sc_agg(nb2, row3, col3, zeros2d)
    return _pool(s, parts, batch2, Wc.T, bc2)
